# baseline (speedup 1.0000x reference)
import functools
import jax
import jax.numpy as jnp
from jax import lax
from jax.experimental import pallas as pl
from jax.experimental.pallas import tpu as pltpu
from jax.experimental.pallas import tpu_sc as plsc

SIZE = 4096
DIM = 512
B = 4
N = 4096
D_EMB = 512

NC = 2
NS = 16
NW = NC * NS
RW = N // NW
CE = 16
NB = 4
AHEAD = 3

_mesh = plsc.VectorSubcoreMesh(core_axis_name="c", subcore_axis_name="s")


@functools.partial(
    pl.kernel,
    out_type=jax.ShapeDtypeStruct((B, N, D_EMB + DIM), jnp.float32),
    mesh=_mesh,
    compiler_params=pltpu.CompilerParams(
        disable_bounds_checks=True, disable_semaphore_checks=True),
    scratch_types=[
        pltpu.VMEM((RW, DIM), jnp.float32),
        pltpu.VMEM((RW,), jnp.int32),
        pltpu.VMEM((NB, CE, D_EMB), jnp.float32),
        pltpu.SemaphoreType.DMA,
        pltpu.SemaphoreType.DMA,
        [pltpu.SemaphoreType.DMA] * NB,
        [pltpu.SemaphoreType.DMA] * NB,
    ],
)
def _sc_pos_embed(emb_hbm, pe_hbm, out_hbm, pe_v, idx_v, emb_v, sem_pe,
                  sem_g, sems_g, sems_s):
    wid = lax.axis_index("s") * NC + lax.axis_index("c")
    base = wid * RW
    n_chunks = B * (RW // CE)
    g_handles = [None] * n_chunks
    s_handles = [None] * n_chunks

    def _gather(j):
        b, c = divmod(j, RW // CE)
        return pltpu.async_copy(
            emb_hbm.at[b, pl.ds(base + c * CE, CE)], emb_v.at[j % NB],
            sems_g[j % NB])

    for j in range(min(AHEAD, n_chunks)):
        g_handles[j] = _gather(j)
    for r in range(RW // 16):
        idx_v[pl.ds(r * 16, 16)] = base + 1 + r * 16 + lax.iota(jnp.int32, 16)
    pltpu.async_copy(pe_hbm.at[idx_v], pe_v, sem_g).wait()
    pe_handles = [
        pltpu.async_copy(
            pe_v, out_hbm.at[b, pl.ds(base, RW), pl.ds(D_EMB, DIM)], sem_pe)
        for b in range(B)
    ]
    for i in range(n_chunks):
        j = i + AHEAD
        if j < n_chunks:
            if j >= NB:
                s_handles[j - NB].wait()
            g_handles[j] = _gather(j)
        g_handles[i].wait()
        b, c = divmod(i, RW // CE)
        s_handles[i] = pltpu.async_copy(
            emb_v.at[i % NB],
            out_hbm.at[b, pl.ds(base + c * CE, CE), pl.ds(0, D_EMB)],
            sems_s[i % NB])
    for i in range(max(0, n_chunks - NB), n_chunks):
        s_handles[i].wait()
    for h in pe_handles:
        h.wait()


def kernel(emb, pe):
    return _sc_pos_embed(emb, pe)


# speedup vs baseline: 1.0517x; 1.0517x over previous
import functools
import jax
import jax.numpy as jnp
from jax import lax
from jax.experimental import pallas as pl
from jax.experimental.pallas import tpu as pltpu
from jax.experimental.pallas import tpu_sc as plsc

SIZE = 4096
DIM = 512
B = 4
N = 4096
D_EMB = 512

NC = 2
NS = 16
NW = NC * NS
RW = N // NW
CE = 16
NB = 4
AHEAD = 2

_mesh = plsc.VectorSubcoreMesh(core_axis_name="c", subcore_axis_name="s")


@functools.partial(
    pl.kernel,
    out_type=jax.ShapeDtypeStruct((B, N, D_EMB + DIM), jnp.float32),
    mesh=_mesh,
    compiler_params=pltpu.CompilerParams(
        disable_bounds_checks=True, disable_semaphore_checks=True),
    scratch_types=[
        pltpu.VMEM((RW, DIM), jnp.float32),
        pltpu.VMEM((RW,), jnp.int32),
        pltpu.VMEM((NB, CE, D_EMB), jnp.float32),
        pltpu.SemaphoreType.DMA,
        pltpu.SemaphoreType.DMA,
        [pltpu.SemaphoreType.DMA] * NB,
        [pltpu.SemaphoreType.DMA] * NB,
    ],
)
def _sc_pos_embed(emb_hbm, pe_hbm, out_hbm, pe_v, idx_v, emb_v, sem_pe,
                  sem_g, sems_g, sems_s):
    wid = lax.axis_index("s") * NC + lax.axis_index("c")
    base = wid * RW
    n_chunks = B * (RW // CE)
    g_handles = [None] * n_chunks
    s_handles = [None] * n_chunks

    def _gather(j):
        b, c = divmod(j, RW // CE)
        return pltpu.async_copy(
            emb_hbm.at[b, pl.ds(base + c * CE, CE)], emb_v.at[j % NB],
            sems_g[j % NB])

    for j in range(min(AHEAD, n_chunks)):
        g_handles[j] = _gather(j)
    for r in range(RW // 16):
        idx_v[pl.ds(r * 16, 16)] = base + 1 + r * 16 + lax.iota(jnp.int32, 16)
    pltpu.async_copy(pe_hbm.at[idx_v], pe_v, sem_g).wait()
    pe_handles = [
        pltpu.async_copy(
            pe_v, out_hbm.at[b, pl.ds(base, RW), pl.ds(D_EMB, DIM)], sem_pe)
        for b in range(B)
    ]
    for i in range(n_chunks):
        j = i + AHEAD
        if j < n_chunks:
            if j >= NB:
                s_handles[j - NB].wait()
            g_handles[j] = _gather(j)
        g_handles[i].wait()
        b, c = divmod(i, RW // CE)
        s_handles[i] = pltpu.async_copy(
            emb_v.at[i % NB],
            out_hbm.at[b, pl.ds(base + c * CE, CE), pl.ds(0, D_EMB)],
            sems_s[i % NB])
    for i in range(max(0, n_chunks - NB), n_chunks):
        s_handles[i].wait()
    for h in pe_handles:
        h.wait()


def kernel(emb, pe):
    return _sc_pos_embed(emb, pe)


# speedup vs baseline: 1.0540x; 1.0022x over previous
import functools
import jax
import jax.numpy as jnp
from jax import lax
from jax.experimental import pallas as pl
from jax.experimental.pallas import tpu as pltpu
from jax.experimental.pallas import tpu_sc as plsc

SIZE = 4096
DIM = 512
B = 4
N = 4096
D_EMB = 512

NC = 2
NS = 16
NW = NC * NS
RW = N // NW
CE = 16
NB = 4
AHEAD = 2

_mesh = plsc.VectorSubcoreMesh(core_axis_name="c", subcore_axis_name="s")


@functools.partial(
    pl.kernel,
    out_type=jax.ShapeDtypeStruct((B, N, D_EMB + DIM), jnp.float32),
    mesh=_mesh,
    compiler_params=pltpu.CompilerParams(
        disable_bounds_checks=True, disable_semaphore_checks=True,
        skip_device_barrier=True),
    scratch_types=[
        pltpu.VMEM((RW, DIM), jnp.float32),
        pltpu.VMEM((RW,), jnp.int32),
        pltpu.VMEM((NB, CE, D_EMB), jnp.float32),
        pltpu.SemaphoreType.DMA,
        pltpu.SemaphoreType.DMA,
        [pltpu.SemaphoreType.DMA] * NB,
        [pltpu.SemaphoreType.DMA] * NB,
    ],
)
def _sc_pos_embed(emb_hbm, pe_hbm, out_hbm, pe_v, idx_v, emb_v, sem_pe,
                  sem_g, sems_g, sems_s):
    wid = lax.axis_index("s") * NC + lax.axis_index("c")
    base = wid * RW
    n_chunks = B * (RW // CE)
    g_handles = [None] * n_chunks
    s_handles = [None] * n_chunks

    def _gather(j):
        b, c = divmod(j, RW // CE)
        return pltpu.async_copy(
            emb_hbm.at[b, pl.ds(base + c * CE, CE)], emb_v.at[j % NB],
            sems_g[j % NB])

    for j in range(min(AHEAD, n_chunks)):
        g_handles[j] = _gather(j)
    for r in range(RW // 16):
        idx_v[pl.ds(r * 16, 16)] = base + 1 + r * 16 + lax.iota(jnp.int32, 16)
    pltpu.async_copy(pe_hbm.at[idx_v], pe_v, sem_g).wait()
    pe_handles = [
        pltpu.async_copy(
            pe_v, out_hbm.at[b, pl.ds(base, RW), pl.ds(D_EMB, DIM)], sem_pe)
        for b in range(B)
    ]
    for i in range(n_chunks):
        j = i + AHEAD
        if j < n_chunks:
            if j >= NB:
                s_handles[j - NB].wait()
            g_handles[j] = _gather(j)
        g_handles[i].wait()
        b, c = divmod(i, RW // CE)
        s_handles[i] = pltpu.async_copy(
            emb_v.at[i % NB],
            out_hbm.at[b, pl.ds(base + c * CE, CE), pl.ds(0, D_EMB)],
            sems_s[i % NB])
    for i in range(max(0, n_chunks - NB), n_chunks):
        s_handles[i].wait()
    for h in pe_handles:
        h.wait()


def kernel(emb, pe):
    return _sc_pos_embed(emb, pe)


# speedup vs baseline: 1.0567x; 1.0025x over previous
import functools
import jax
import jax.numpy as jnp
from jax import lax
from jax.experimental import pallas as pl
from jax.experimental.pallas import tpu as pltpu
from jax.experimental.pallas import tpu_sc as plsc

SIZE = 4096
DIM = 512
B = 4
N = 4096
D_EMB = 512

NC = 2
NS = 16
NW = NC * NS
RW = N // NW
CE = 16
NB = 4
AHEAD = 2

_mesh = plsc.VectorSubcoreMesh(core_axis_name="c", subcore_axis_name="s")


@functools.partial(
    pl.kernel,
    out_type=jax.ShapeDtypeStruct((B, N, D_EMB + DIM), jnp.float32),
    mesh=_mesh,
    compiler_params=pltpu.CompilerParams(
        disable_bounds_checks=True, disable_semaphore_checks=True),
    scratch_types=[
        pltpu.VMEM((RW, DIM), jnp.float32),
        pltpu.VMEM((RW,), jnp.int32),
        pltpu.VMEM((NB, CE, D_EMB), jnp.float32),
        pltpu.SemaphoreType.DMA,
        pltpu.SemaphoreType.DMA,
        [pltpu.SemaphoreType.DMA] * NB,
        [pltpu.SemaphoreType.DMA] * NB,
    ],
)
def _sc_pos_embed(emb_hbm, pe_hbm, out_hbm, pe_v, idx_v, emb_v, sem_pe,
                  sem_g, sems_g, sems_s):
    wid = lax.axis_index("s") * NC + lax.axis_index("c")
    base = wid * RW
    n_chunks = B * (RW // CE)
    g_handles = [None] * n_chunks
    s_handles = [None] * n_chunks

    def _gather(j):
        b, c = divmod(j, RW // CE)
        return pltpu.async_copy(
            emb_hbm.at[b, pl.ds(base + c * CE, CE)], emb_v.at[j % NB],
            sems_g[j % NB])

    for j in range(min(AHEAD, n_chunks)):
        g_handles[j] = _gather(j)
    for r in range(RW // 16):
        idx_v[pl.ds(r * 16, 16)] = base + 1 + r * 16 + lax.iota(jnp.int32, 16)
    pltpu.async_copy(pe_hbm.at[idx_v], pe_v, sem_g).wait()
    pe_handles = [
        pltpu.async_copy(
            pe_v, out_hbm.at[b, pl.ds(base, RW), pl.ds(D_EMB, DIM)], sem_pe)
        for b in range(B)
    ]
    for i in range(n_chunks):
        j = i + AHEAD
        if j < n_chunks:
            if j >= NB:
                s_handles[j - NB].wait()
            g_handles[j] = _gather(j)
        g_handles[i].wait()
        b, c = divmod(i, RW // CE)
        s_handles[i] = pltpu.async_copy(
            emb_v.at[i % NB],
            out_hbm.at[b, pl.ds(base + c * CE, CE), pl.ds(0, D_EMB)],
            sems_s[i % NB])
    for i in range(max(0, n_chunks - NB), n_chunks):
        s_handles[i].wait()
    for h in pe_handles:
        h.wait()


def kernel(emb, pe):
    return _sc_pos_embed(emb, pe)
